# Initial kernel scaffold; baseline (speedup 1.0000x reference)
import jax
import jax.numpy as jnp
from jax.experimental import pallas as pl


def kernel(x, edge_index, W1_l, W1_r, b1, W2_l, W2_r, b2):
    raise NotImplementedError("write your pallas kernel here")


# speedup vs baseline: 3.8690x; 3.8690x over previous
import functools

import jax
import jax.numpy as jnp
from jax import lax
from jax.experimental import pallas as pl
from jax.experimental.pallas import tpu as pltpu
from jax.experimental.pallas import tpu_sc as plsc

N_NODES = 10000
N_EDGES = 320000
D = 128

NC = 2
NS = 16
NW = NC * NS

B = 128
K = 80
EPT = B * K
E_PAD = NW * EPT
ACC_N = 10240
SLAB = ACC_N // NS

def _seg_body(x_hbm, src_hbm, dst_hbm, out_hbm, cnt_hbm,
              rows_v, src_v, dst_v, ones_v, zc_v, acc_sh, cnt_sh, sem):
    _ZERO16 = jnp.zeros((16,), jnp.float32)
    _ONES16 = jnp.ones((16,), jnp.float32)
    cid = lax.axis_index("c")
    sid = lax.axis_index("s")
    wid = sid * NC + cid

    def zr(i, _):
        rows_v[i // 8, pl.ds((i % 8) * 16, 16)] = _ZERO16
        return 0
    lax.fori_loop(0, B * 8, zr, 0)
    base = sid * SLAB
    for k in range(SLAB // B):
        pltpu.sync_copy(rows_v, acc_sh.at[pl.ds(base + k * B, B)])

    def zc(i, _):
        ones_v[pl.ds(i * 16, 16)] = _ONES16
        return 0
    lax.fori_loop(0, B // 16, zc, 0)

    def zs(i, _):
        zc_v[pl.ds(i * 16, 16)] = _ZERO16
        return 0
    lax.fori_loop(0, SLAB // 16, zs, 0)
    pltpu.sync_copy(zc_v, cnt_sh.at[pl.ds(sid * SLAB, SLAB)])

    pltpu.sync_copy(src_hbm.at[wid], src_v)
    pltpu.sync_copy(dst_hbm.at[wid], dst_v)
    plsc.subcore_barrier()

    def body(j, _):
        pltpu.async_copy(x_hbm.at[src_v.at[j]], rows_v, sem).wait()
        pltpu.sync_copy(ones_v, cnt_sh.at[dst_v.at[j]], add=True)
        pltpu.sync_copy(rows_v, acc_sh.at[dst_v.at[j]], add=True)
        return 0
    lax.fori_loop(0, K, body, 0)

    plsc.subcore_barrier()

    for k in range(SLAB // B):
        pltpu.sync_copy(acc_sh.at[pl.ds(base + k * B, B)], rows_v)
        pltpu.sync_copy(rows_v, out_hbm.at[cid, pl.ds(base + k * B, B)])
    pltpu.sync_copy(cnt_sh.at[pl.ds(sid * SLAB, SLAB)], zc_v)
    pltpu.sync_copy(zc_v, cnt_hbm.at[cid, pl.ds(sid * SLAB, SLAB)])


_seg_sum = functools.partial(
    pl.kernel,
    out_type=(
        jax.ShapeDtypeStruct((NC, ACC_N, D), jnp.float32),
        jax.ShapeDtypeStruct((NC, ACC_N), jnp.float32),
    ),
    mesh=plsc.VectorSubcoreMesh(core_axis_name="c", subcore_axis_name="s"),
    scratch_types=[
        pltpu.VMEM((B, D), jnp.float32),
        pltpu.VMEM((K, B), jnp.int32),
        pltpu.VMEM((K, B), jnp.int32),
        pltpu.VMEM((B,), jnp.float32),
        pltpu.VMEM((SLAB,), jnp.float32),
        pltpu.VMEM_SHARED((ACC_N, D), jnp.float32),
        pltpu.VMEM_SHARED((ACC_N,), jnp.float32),
        pltpu.SemaphoreType.DMA,
    ],
)(_seg_body)


def _dense_body(relu, p_ref, cnt_ref, x_ref, wl_ref, wr_ref, b_ref, o_ref):
    s = p_ref[0] + p_ref[1]
    c = cnt_ref[0] + cnt_ref[1]
    agg = s * (1.0 / jnp.maximum(c, 1.0))[:, None]
    y = (jnp.dot(agg, wl_ref[...], preferred_element_type=jnp.float32)
         + jnp.dot(x_ref[...], wr_ref[...], preferred_element_type=jnp.float32)
         + b_ref[...])
    o_ref[...] = jnp.maximum(y, 0.0) if relu else y


def _dense(p, cnt, x, wl, wr, b, relu):
    RB = 512
    grid = ACC_N // RB
    return pl.pallas_call(
        functools.partial(_dense_body, relu),
        grid=(grid,),
        in_specs=[
            pl.BlockSpec((NC, RB, D), lambda i: (0, i, 0)),
            pl.BlockSpec((NC, RB), lambda i: (0, i)),
            pl.BlockSpec((RB, D), lambda i: (i, 0)),
            pl.BlockSpec((D, D), lambda i: (0, 0)),
            pl.BlockSpec((D, D), lambda i: (0, 0)),
            pl.BlockSpec((1, D), lambda i: (0, 0)),
        ],
        out_specs=pl.BlockSpec((RB, D), lambda i: (i, 0)),
        out_shape=jax.ShapeDtypeStruct((ACC_N, D), jnp.float32),
    )(p, cnt, x, wl, wr, b)


def kernel(x, edge_index, W1_l, W1_r, b1, W2_l, W2_r, b2):
    src = edge_index[0].astype(jnp.int32)
    dst = edge_index[1].astype(jnp.int32)
    pad = E_PAD - N_EDGES
    src_p = jnp.concatenate(
        [src, jnp.zeros((pad,), jnp.int32)]).reshape(NW, K, B)
    dst_p = jnp.concatenate(
        [dst, jnp.full((pad,), N_NODES, jnp.int32)]).reshape(NW, K, B)
    x_pad = jnp.concatenate(
        [x, jnp.zeros((ACC_N - N_NODES, D), x.dtype)], axis=0)
    b1r = b1.reshape(1, D)
    b2r = b2.reshape(1, D)

    p1, cnt = _seg_sum(x_pad, src_p, dst_p)
    h = _dense(p1, cnt, x_pad, W1_l, W1_r, b1r, relu=True)
    p2, _ = _seg_sum(h, src_p, dst_p)
    out = _dense(p2, cnt, h, W2_l, W2_r, b2r, relu=False)
    return out[:N_NODES]
